# baseline (speedup 1.0000x reference)
import functools

import jax
import jax.numpy as jnp
from jax import lax
from jax.experimental import pallas as pl
from jax.experimental.pallas import tpu as pltpu
from jax.experimental.pallas import tpu_sc as plsc

M = 400
KK = 10
BETA = 0.15
D = 256
QN = 64

_PREC_REF = lax.Precision.DEFAULT


def _split3(x):
    hi = x.astype(jnp.bfloat16).astype(jnp.float32)
    r1 = x - hi
    mid = r1.astype(jnp.bfloat16).astype(jnp.float32)
    return hi, mid, r1 - mid


def _split2(x):
    hi = x.astype(jnp.bfloat16).astype(jnp.float32)
    return hi, x - hi
_F32 = jnp.float32


_NC, _NS = 2, 16
_NW = _NC * _NS
_B = QN * M
_BPW = _B // _NW
_CH = 80
_NCHUNK = _BPW // _CH


def _sc_gather_body(x_hbm, idx_hbm, out_hbm, idx_v, rows_v, sem):
    wid = lax.axis_index("s") * _NC + lax.axis_index("c")
    base = wid * _BPW
    for c in range(_NCHUNK):
        off = base + c * _CH
        pltpu.sync_copy(idx_hbm.at[pl.ds(off, _CH)], idx_v)
        pltpu.async_copy(x_hbm.at[idx_v], rows_v, sem).wait()
        pltpu.sync_copy(rows_v, out_hbm.at[pl.ds(off, _CH)])


@functools.cache
def _sc_gather():
    return pl.kernel(
        _sc_gather_body,
        out_type=jax.ShapeDtypeStruct((_B, D), _F32),
        mesh=plsc.VectorSubcoreMesh(
            core_axis_name="c", subcore_axis_name="s",
            num_cores=_NC, num_subcores=_NS),
        scratch_types=[
            pltpu.VMEM((_CH,), jnp.int32),
            pltpu.VMEM((_CH, D), _F32),
            pltpu.SemaphoreType.DMA,
        ],
    )


def _tc_body(g_ref, q_ref, idx_ref, rerank_ref, res_ref, pre_ref, xdba_ref):
    a = g_ref[0]
    q = q_ref[0]
    idxr = idx_ref[0]

    iota_j = lax.broadcasted_iota(jnp.int32, (M, M), 1)
    iota_i = lax.broadcasted_iota(jnp.int32, (M, M), 0)

    s = lax.dot_general(a, a, (((1,), (1,)), ((), ())),
                        precision=_PREC_REF, preferred_element_type=_F32)

    selsT, ws_row = [], []
    scur = s
    for k in range(KK):
        m = jnp.max(scur, axis=0, keepdims=True)
        am = jnp.min(jnp.where(scur == m, iota_i, M), axis=0,
                     keepdims=True)
        sel = iota_i == am
        selsT.append(sel)
        if k > 0:
            ws_row.append(BETA * m)
        if k < KK - 1:
            scur = jnp.where(sel, -jnp.inf, scur)

    eye = (iota_i == iota_j).astype(_F32)
    w27 = jnp.concatenate(_split3(jnp.concatenate(ws_row, axis=0)), axis=0)
    wt = lax.dot_general(eye, w27, (((1,), (1,)), ((), ())),
                         precision=_PREC_REF, preferred_element_type=_F32)
    w_col = [None] + [(wt[:, k:k + 1] + wt[:, 9 + k:10 + k]) + wt[:, 18 + k:19 + k]
                      for k in range(KK - 1)]

    a_split = jnp.concatenate(_split3(a), axis=1)
    ts, ds = [], []
    for k in range(KK):
        e3 = lax.dot_general(selsT[k].astype(_F32), a_split,
                             (((0,), (0,)), ((), ())),
                             precision=_PREC_REF, preferred_element_type=_F32)
        ext = (e3[:, :D] + e3[:, D:2 * D]) + e3[:, 2 * D:]
        ts.append(ext if k == 0 else w_col[k] * ext)
        ds.append(jnp.ones((M, 1), _F32) if k == 0 else w_col[k])

    def _tree_sum(vals):
        u = [vals[k] + vals[k + 8] if k + 8 < len(vals) else vals[k]
             for k in range(8)]
        v = [u[k] + u[k + 4] for k in range(4)]
        w = [v[k] + v[k + 2] for k in range(2)]
        return w[0] + w[1]

    xd = _tree_sum(ts) / _tree_sum(ds)
    xdba_ref[0] = xd

    r = lax.dot_general(q, xd, (((1,), (1,)), ((), ())),
                        precision=_PREC_REF, preferred_element_type=_F32)
    res_ref[0] = r

    r_split = jnp.concatenate(_split3(r), axis=0)
    rc3 = lax.dot_general(eye, r_split, (((1,), (1,)), ((), ())),
                          precision=_PREC_REF, preferred_element_type=_F32)
    rcol = (rc3[:, 0:1] + rc3[:, 1:2]) + rc3[:, 2:3]

    gt = (r > rcol).astype(_F32)
    eql = ((r == rcol) & (iota_j < iota_i)).astype(_F32)
    ones_col = jnp.ones((M, 1), _F32)
    rank_f = lax.dot_general(gt + eql, ones_col, (((1,), (0,)), ((), ())),
                             precision=_PREC_REF, preferred_element_type=_F32)
    rank = jnp.round(rank_f).astype(jnp.int32)

    perm = (rank == iota_j).astype(_F32)
    ivals = lax.broadcasted_iota(jnp.int32, (1, M), 1).astype(_F32)
    ih, il = _split2(ivals)
    rh, rl = _split2(idxr.astype(_F32))
    vrows = jnp.concatenate([ih, il, rh, rl], axis=0)
    o4 = lax.dot_general(vrows, perm, (((1,), (0,)), ((), ())),
                         precision=_PREC_REF, preferred_element_type=_F32)
    pre_row = o4[0:1, :] + o4[1:2, :]
    rer_row = o4[2:3, :] + o4[3:4, :]
    pre_ref[0] = jnp.round(pre_row).astype(jnp.int32)
    rerank_ref[0] = jnp.round(rer_row).astype(jnp.int32)


def _tc_compute(g3, q3, idx3):
    return pl.pallas_call(
        _tc_body,
        grid=(QN,),
        in_specs=[
            pl.BlockSpec((1, M, D), lambda i: (i, 0, 0)),
            pl.BlockSpec((1, 1, D), lambda i: (i, 0, 0)),
            pl.BlockSpec((1, 1, M), lambda i: (i, 0, 0)),
        ],
        out_specs=[
            pl.BlockSpec((1, 1, M), lambda i: (i, 0, 0)),
            pl.BlockSpec((1, 1, M), lambda i: (i, 0, 0)),
            pl.BlockSpec((1, 1, M), lambda i: (i, 0, 0)),
            pl.BlockSpec((1, M, D), lambda i: (i, 0, 0)),
        ],
        out_shape=[
            jax.ShapeDtypeStruct((QN, 1, M), jnp.int32),
            jax.ShapeDtypeStruct((QN, 1, M), _F32),
            jax.ShapeDtypeStruct((QN, 1, M), jnp.int32),
            jax.ShapeDtypeStruct((QN, M, D), _F32),
        ],
    )(g3, q3, idx3)


def kernel(X, Q, ranks):
    idx = ranks[:M, :].T.astype(jnp.int32)
    g = _sc_gather()(X, idx.reshape(-1))
    g3 = g.reshape(QN, M, D)
    rerank3, res3, pre3, xdba = _tc_compute(
        g3, Q.reshape(QN, 1, D), idx.reshape(QN, 1, M))
    return (rerank3.reshape(QN, M), res3.reshape(QN, M),
            pre3.reshape(QN, M), xdba)


# speedup vs baseline: 1.3166x; 1.3166x over previous
import functools

import jax
import jax.numpy as jnp
from jax import lax
from jax.experimental import pallas as pl
from jax.experimental.pallas import tpu as pltpu
from jax.experimental.pallas import tpu_sc as plsc

M = 400
KK = 10
BETA = 0.15
D = 256
QN = 64

_PREC_REF = lax.Precision.DEFAULT


def _split3(x):
    hi = x.astype(jnp.bfloat16).astype(jnp.float32)
    r1 = x - hi
    mid = r1.astype(jnp.bfloat16).astype(jnp.float32)
    return hi, mid, r1 - mid


def _split2(x):
    hi = x.astype(jnp.bfloat16).astype(jnp.float32)
    return hi, x - hi
_F32 = jnp.float32


_NC, _NS = 2, 16
_NW = _NC * _NS
_B = QN * M
_BPW = _B // _NW
_CH = 80
_NCHUNK = _BPW // _CH


def _sc_gather_body(x_hbm, idx_hbm, out_hbm, idx_v, rows_v, sem):
    wid = lax.axis_index("s") * _NC + lax.axis_index("c")
    base = wid * _BPW
    for c in range(_NCHUNK):
        off = base + c * _CH
        pltpu.sync_copy(idx_hbm.at[pl.ds(off, _CH)], idx_v)
        pltpu.async_copy(x_hbm.at[idx_v], rows_v, sem).wait()
        pltpu.sync_copy(rows_v, out_hbm.at[pl.ds(off, _CH)])


@functools.cache
def _sc_gather():
    return pl.kernel(
        _sc_gather_body,
        out_type=jax.ShapeDtypeStruct((_B, D), _F32),
        mesh=plsc.VectorSubcoreMesh(
            core_axis_name="c", subcore_axis_name="s",
            num_cores=_NC, num_subcores=_NS),
        scratch_types=[
            pltpu.VMEM((_CH,), jnp.int32),
            pltpu.VMEM((_CH, D), _F32),
            pltpu.SemaphoreType.DMA,
        ],
    )


_QB = 2


def _tc_body(g_ref, q_ref, idx_ref, rerank_ref, res_ref, pre_ref, xdba_ref):
    for t in range(_QB):
        rer_row, r, pre_row, xd = _one_query(g_ref[t], q_ref[t], idx_ref[t])
        rerank_ref[t] = rer_row
        res_ref[t] = r
        pre_ref[t] = pre_row
        xdba_ref[t] = xd


def _one_query(a, q, idxr):
    iota_j = lax.broadcasted_iota(jnp.int32, (M, M), 1)
    iota_i = lax.broadcasted_iota(jnp.int32, (M, M), 0)

    s = lax.dot_general(a, a, (((1,), (1,)), ((), ())),
                        precision=_PREC_REF, preferred_element_type=_F32)

    sels, ws = [], []
    scur = s
    for k in range(KK):
        m = jnp.max(scur, axis=1, keepdims=True)
        am = jnp.min(jnp.where(scur == m, iota_j, M), axis=1,
                     keepdims=True)
        sel = iota_j == am
        sels.append(sel)
        ws.append(jnp.ones((M, 1), _F32) if k == 0 else BETA * m)
        if k < KK - 1:
            scur = jnp.where(sel, -jnp.inf, scur)

    a_split = jnp.concatenate(_split3(a), axis=1)
    ts, ds = [], []
    for k in range(KK):
        e3 = lax.dot_general(sels[k].astype(_F32), a_split,
                             (((1,), (0,)), ((), ())),
                             precision=_PREC_REF, preferred_element_type=_F32)
        ext = (e3[:, :D] + e3[:, D:2 * D]) + e3[:, 2 * D:]
        ts.append(ext if k == 0 else ws[k] * ext)
        ds.append(ws[k])

    def _tree_sum(vals):
        u = [vals[k] + vals[k + 8] if k + 8 < len(vals) else vals[k]
             for k in range(8)]
        v = [u[k] + u[k + 4] for k in range(4)]
        w = [v[k] + v[k + 2] for k in range(2)]
        return w[0] + w[1]

    xd = _tree_sum(ts) / _tree_sum(ds)

    r = lax.dot_general(q, xd, (((1,), (1,)), ((), ())),
                        precision=_PREC_REF, preferred_element_type=_F32)

    eye = (iota_i == iota_j).astype(_F32)
    r_split = jnp.concatenate(_split3(r), axis=0)
    rc3 = lax.dot_general(eye, r_split, (((1,), (1,)), ((), ())),
                          precision=_PREC_REF, preferred_element_type=_F32)
    rcol = (rc3[:, 0:1] + rc3[:, 1:2]) + rc3[:, 2:3]

    gt = (r > rcol).astype(_F32)
    eql = ((r == rcol) & (iota_j < iota_i)).astype(_F32)
    ones_col = jnp.ones((M, 1), _F32)
    rank_f = lax.dot_general(gt + eql, ones_col, (((1,), (0,)), ((), ())),
                             precision=_PREC_REF, preferred_element_type=_F32)
    rank = jnp.round(rank_f).astype(jnp.int32)

    perm = (rank == iota_j).astype(_F32)
    ivals = lax.broadcasted_iota(jnp.int32, (1, M), 1).astype(_F32)
    ih, il = _split2(ivals)
    rh, rl = _split2(idxr.astype(_F32))
    vrows = jnp.concatenate([ih, il, rh, rl], axis=0)
    o4 = lax.dot_general(vrows, perm, (((1,), (0,)), ((), ())),
                         precision=_PREC_REF, preferred_element_type=_F32)
    pre_row = o4[0:1, :] + o4[1:2, :]
    rer_row = o4[2:3, :] + o4[3:4, :]
    return (jnp.round(rer_row).astype(jnp.int32), r,
            jnp.round(pre_row).astype(jnp.int32), xd)


def _tc_compute(g3, q3, idx3):
    return pl.pallas_call(
        _tc_body,
        grid=(QN // _QB,),
        in_specs=[
            pl.BlockSpec((_QB, M, D), lambda i: (i, 0, 0)),
            pl.BlockSpec((_QB, 1, D), lambda i: (i, 0, 0)),
            pl.BlockSpec((_QB, 1, M), lambda i: (i, 0, 0)),
        ],
        out_specs=[
            pl.BlockSpec((_QB, 1, M), lambda i: (i, 0, 0)),
            pl.BlockSpec((_QB, 1, M), lambda i: (i, 0, 0)),
            pl.BlockSpec((_QB, 1, M), lambda i: (i, 0, 0)),
            pl.BlockSpec((_QB, M, D), lambda i: (i, 0, 0)),
        ],
        out_shape=[
            jax.ShapeDtypeStruct((QN, 1, M), jnp.int32),
            jax.ShapeDtypeStruct((QN, 1, M), _F32),
            jax.ShapeDtypeStruct((QN, 1, M), jnp.int32),
            jax.ShapeDtypeStruct((QN, M, D), _F32),
        ],
    )(g3, q3, idx3)


def kernel(X, Q, ranks):
    idx = ranks[:M, :].T.astype(jnp.int32)
    g = _sc_gather()(X, idx.reshape(-1))
    g3 = g.reshape(QN, M, D)
    rerank3, res3, pre3, xdba = _tc_compute(
        g3, Q.reshape(QN, 1, D), idx.reshape(QN, 1, M))
    return (rerank3.reshape(QN, M), res3.reshape(QN, M),
            pre3.reshape(QN, M), xdba)


# speedup vs baseline: 1.3684x; 1.0393x over previous
import functools

import jax
import jax.numpy as jnp
from jax import lax
from jax.experimental import pallas as pl
from jax.experimental.pallas import tpu as pltpu
from jax.experimental.pallas import tpu_sc as plsc

M = 400
KK = 10
BETA = 0.15
D = 256
QN = 64

_PREC_REF = lax.Precision.DEFAULT


def _split3(x):
    hi = x.astype(jnp.bfloat16).astype(jnp.float32)
    r1 = x - hi
    mid = r1.astype(jnp.bfloat16).astype(jnp.float32)
    return hi, mid, r1 - mid


def _split2(x):
    hi = x.astype(jnp.bfloat16).astype(jnp.float32)
    return hi, x - hi
_F32 = jnp.float32


_NC, _NS = 2, 16
_NW = _NC * _NS
_B = QN * M
_BPW = _B // _NW
_CH = 80
_NCHUNK = _BPW // _CH


def _sc_gather_body(x_hbm, idx_hbm, out_hbm, idx_v, rows_a, rows_b,
                    sem_g, sem_w):
    wid = lax.axis_index("s") * _NC + lax.axis_index("c")
    base = wid * _BPW
    pltpu.sync_copy(idx_hbm.at[wid], idx_v)
    bufs = (rows_a, rows_b)
    gh = [None] * _NCHUNK
    wh = [None] * _NCHUNK
    gh[0] = pltpu.async_copy(x_hbm.at[idx_v.at[0]], bufs[0], sem_g)
    for c in range(_NCHUNK):
        buf = bufs[c % 2]
        gh[c].wait()
        if c + 1 < _NCHUNK:
            if c >= 1:
                wh[c - 1].wait()
            gh[c + 1] = pltpu.async_copy(
                x_hbm.at[idx_v.at[c + 1]], bufs[(c + 1) % 2], sem_g)
        wh[c] = pltpu.async_copy(
            buf, out_hbm.at[pl.ds(base + c * _CH, _CH)], sem_w)
    wh[_NCHUNK - 1].wait()


@functools.cache
def _sc_gather():
    return pl.kernel(
        _sc_gather_body,
        out_type=jax.ShapeDtypeStruct((_B, D), _F32),
        mesh=plsc.VectorSubcoreMesh(
            core_axis_name="c", subcore_axis_name="s",
            num_cores=_NC, num_subcores=_NS),
        scratch_types=[
            pltpu.VMEM((_NCHUNK, _CH), jnp.int32),
            pltpu.VMEM((_CH, D), _F32),
            pltpu.VMEM((_CH, D), _F32),
            pltpu.SemaphoreType.DMA,
            pltpu.SemaphoreType.DMA,
        ],
    )


_QB = 2


def _tc_body(g_ref, q_ref, idx_ref, rerank_ref, res_ref, pre_ref, xdba_ref):
    for t in range(_QB):
        rer_row, r, pre_row, xd = _one_query(g_ref[t], q_ref[t], idx_ref[t])
        rerank_ref[t] = rer_row
        res_ref[t] = r
        pre_ref[t] = pre_row
        xdba_ref[t] = xd


def _one_query(a, q, idxr):
    iota_j = lax.broadcasted_iota(jnp.int32, (M, M), 1)
    iota_i = lax.broadcasted_iota(jnp.int32, (M, M), 0)

    s = lax.dot_general(a, a, (((1,), (1,)), ((), ())),
                        precision=_PREC_REF, preferred_element_type=_F32)

    sels, ws = [], []
    scur = s
    for k in range(KK):
        m = jnp.max(scur, axis=1, keepdims=True)
        am = jnp.min(jnp.where(scur == m, iota_j, M), axis=1,
                     keepdims=True)
        sel = iota_j == am
        sels.append(sel)
        ws.append(jnp.ones((M, 1), _F32) if k == 0 else BETA * m)
        if k < KK - 1:
            scur = jnp.where(sel, -jnp.inf, scur)

    a_split = jnp.concatenate(_split3(a), axis=1)
    ts, ds = [], []
    for k in range(KK):
        e3 = lax.dot_general(sels[k].astype(_F32), a_split,
                             (((1,), (0,)), ((), ())),
                             precision=_PREC_REF, preferred_element_type=_F32)
        ext = (e3[:, :D] + e3[:, D:2 * D]) + e3[:, 2 * D:]
        ts.append(ext if k == 0 else ws[k] * ext)
        ds.append(ws[k])

    def _tree_sum(vals):
        u = [vals[k] + vals[k + 8] if k + 8 < len(vals) else vals[k]
             for k in range(8)]
        v = [u[k] + u[k + 4] for k in range(4)]
        w = [v[k] + v[k + 2] for k in range(2)]
        return w[0] + w[1]

    xd = _tree_sum(ts) / _tree_sum(ds)

    r = lax.dot_general(q, xd, (((1,), (1,)), ((), ())),
                        precision=_PREC_REF, preferred_element_type=_F32)

    eye = (iota_i == iota_j).astype(_F32)
    r_split = jnp.concatenate(_split3(r), axis=0)
    rc3 = lax.dot_general(eye, r_split, (((1,), (1,)), ((), ())),
                          precision=_PREC_REF, preferred_element_type=_F32)
    rcol = (rc3[:, 0:1] + rc3[:, 1:2]) + rc3[:, 2:3]

    gt = (r > rcol).astype(_F32)
    eql = ((r == rcol) & (iota_j < iota_i)).astype(_F32)
    ones_col = jnp.ones((M, 1), _F32)
    rank_f = lax.dot_general(gt + eql, ones_col, (((1,), (0,)), ((), ())),
                             precision=_PREC_REF, preferred_element_type=_F32)
    rank = jnp.round(rank_f).astype(jnp.int32)

    perm = (rank == iota_j).astype(_F32)
    ivals = lax.broadcasted_iota(jnp.int32, (1, M), 1).astype(_F32)
    ih, il = _split2(ivals)
    rh, rl = _split2(idxr.astype(_F32))
    vrows = jnp.concatenate([ih, il, rh, rl], axis=0)
    o4 = lax.dot_general(vrows, perm, (((1,), (0,)), ((), ())),
                         precision=_PREC_REF, preferred_element_type=_F32)
    pre_row = o4[0:1, :] + o4[1:2, :]
    rer_row = o4[2:3, :] + o4[3:4, :]
    return (jnp.round(rer_row).astype(jnp.int32), r,
            jnp.round(pre_row).astype(jnp.int32), xd)


def _tc_compute(g3, q3, idx3):
    return pl.pallas_call(
        _tc_body,
        grid=(QN // _QB,),
        in_specs=[
            pl.BlockSpec((_QB, M, D), lambda i: (i, 0, 0)),
            pl.BlockSpec((_QB, 1, D), lambda i: (i, 0, 0)),
            pl.BlockSpec((_QB, 1, M), lambda i: (i, 0, 0)),
        ],
        out_specs=[
            pl.BlockSpec((_QB, 1, M), lambda i: (i, 0, 0)),
            pl.BlockSpec((_QB, 1, M), lambda i: (i, 0, 0)),
            pl.BlockSpec((_QB, 1, M), lambda i: (i, 0, 0)),
            pl.BlockSpec((_QB, M, D), lambda i: (i, 0, 0)),
        ],
        out_shape=[
            jax.ShapeDtypeStruct((QN, 1, M), jnp.int32),
            jax.ShapeDtypeStruct((QN, 1, M), _F32),
            jax.ShapeDtypeStruct((QN, 1, M), jnp.int32),
            jax.ShapeDtypeStruct((QN, M, D), _F32),
        ],
    )(g3, q3, idx3)


def kernel(X, Q, ranks):
    idx = ranks[:M, :].T.astype(jnp.int32)
    g = _sc_gather()(X, idx.reshape(_NW, _NCHUNK, _CH))
    g3 = g.reshape(QN, M, D)
    rerank3, res3, pre3, xdba = _tc_compute(
        g3, Q.reshape(QN, 1, D), idx.reshape(QN, 1, M))
    return (rerank3.reshape(QN, M), res3.reshape(QN, M),
            pre3.reshape(QN, M), xdba)
